# baseline (speedup 1.0000x reference)
import functools

import jax
import jax.numpy as jnp
from jax import lax
from jax.experimental import pallas as pl
from jax.experimental.pallas import tpu as pltpu
from jax.experimental.pallas import tpu_sc as plsc

_R = 8192
_C = 512

_R_SC = 2048
_R_TC = _R - _R_SC

_W = 4
_BLK = 256

_NW = 32
_LANES = 16
_SC_TOTAL = _R_SC * _C
_SC_CHUNK = _SC_TOTAL // _NW
_SC_BLOCK = 8192
_SC_NB = _SC_CHUNK // _SC_BLOCK
_SC_OFF = _R_TC * _C


def _tc_body(*refs):
    p_refs = refs[:_W]
    t_refs = refs[_W:2 * _W]
    out_ref = refs[2 * _W]
    err_acc, cnt_acc = refs[2 * _W + 1], refs[2 * _W + 2]
    i = pl.program_id(0)

    @pl.when(i == 0)
    def _init():
        err_acc[...] = jnp.zeros_like(err_acc)
        cnt_acc[...] = jnp.zeros_like(cnt_acc)

    e = jnp.zeros((1, _C), jnp.float32)
    c = jnp.zeros((1, _C), jnp.float32)
    for p_ref, t_ref in zip(p_refs, t_refs):
        p = p_ref[...]
        t = t_ref[...]
        valid = t > 0.0
        err = jnp.where(valid, jnp.abs(p - t), 0.0)
        cnt = jnp.where(valid, 1.0, 0.0)
        e += jnp.sum(err, axis=0, keepdims=True)
        c += jnp.sum(cnt, axis=0, keepdims=True)
    err_acc[...] += e
    cnt_acc[...] += c

    @pl.when(i == pl.num_programs(0) - 1)
    def _fini():
        out_ref[0, 0] = jnp.sum(err_acc[...])
        out_ref[0, 1] = jnp.sum(cnt_acc[...])


def _tc_partial(p, t):
    steps = _R_TC // _W // _BLK
    specs = [
        pl.BlockSpec((_BLK, _C), lambda i, w=w: (i + w * steps, 0))
        for w in range(_W)
    ]
    return pl.pallas_call(
        _tc_body,
        grid=(steps,),
        in_specs=specs + specs,
        out_specs=pl.BlockSpec(memory_space=pltpu.SMEM),
        out_shape=jax.ShapeDtypeStruct((1, 2), jnp.float32),
        scratch_shapes=[
            pltpu.VMEM((1, _C), jnp.float32),
            pltpu.VMEM((1, _C), jnp.float32),
        ],
    )(*([p] * _W + [t] * _W))


def _sc_kernel_body(p_hbm, t_hbm, out_hbm, p_buf, t_buf, e_acc, c_acc):
    wid = lax.axis_index("s") * 2 + lax.axis_index("c")
    base = _SC_OFF + wid * _SC_CHUNK

    e_acc[...] = jnp.zeros((_LANES,), jnp.float32)
    c_acc[...] = jnp.zeros((_LANES,), jnp.float32)

    for b in range(_SC_NB):
        pltpu.sync_copy(p_hbm.at[pl.ds(base + b * _SC_BLOCK, _SC_BLOCK)], p_buf)
        pltpu.sync_copy(t_hbm.at[pl.ds(base + b * _SC_BLOCK, _SC_BLOCK)], t_buf)

        @pl.loop(0, _SC_BLOCK, step=8 * _LANES)
        def _(i):
            for k in range(8):
                sl = pl.ds(i + k * _LANES, _LANES)
                p = p_buf[sl]
                t = t_buf[sl]
                valid = t > 0.0
                err = jnp.where(valid, jnp.abs(p - t), 0.0)
                cnt = jnp.where(valid, 1.0, 0.0)
                plsc.addupdate(e_acc.at[:], err)
                plsc.addupdate(c_acc.at[:], cnt)

    pltpu.sync_copy(e_acc, out_hbm.at[0, wid])
    pltpu.sync_copy(c_acc, out_hbm.at[1, wid])


def _sc_partial(p_flat, t_flat):
    mesh = plsc.VectorSubcoreMesh(core_axis_name="c", subcore_axis_name="s")
    k = pl.kernel(
        _sc_kernel_body,
        out_type=jax.ShapeDtypeStruct((2, _NW, _LANES), jnp.float32),
        mesh=mesh,
        scratch_types=[
            pltpu.VMEM((_SC_BLOCK,), jnp.float32),
            pltpu.VMEM((_SC_BLOCK,), jnp.float32),
            pltpu.VMEM((_LANES,), jnp.float32),
            pltpu.VMEM((_LANES,), jnp.float32),
        ],
    )
    return k(p_flat, t_flat)


def kernel(pred, target):
    p = pred.reshape(_R, _C)
    t = target.reshape(_R, _C)
    tc = _tc_partial(p, t)
    sc = _sc_partial(pred.reshape(-1), target.reshape(-1))
    s = tc[0, 0] + jnp.sum(sc[0])
    n = tc[0, 1] + jnp.sum(sc[1])
    mae = s / jnp.maximum(n, 1.0)
    return jnp.where(n < 10.0, jnp.float32(-1.0), mae)


# speedup vs baseline: 1.8766x; 1.8766x over previous
import jax
import jax.numpy as jnp
from jax import lax
from jax.experimental import pallas as pl
from jax.experimental.pallas import tpu as pltpu
from jax.experimental.pallas import tpu_sc as plsc

_R = 8192
_C = 512

_R_SC = 2048
_R_TC = _R - _R_SC

_W = 4
_BLK = 256

_NW = 32
_LANES = 16
_SC_ROWS = _R_SC // _NW
_SC_BR = 16
_SC_NB = _SC_ROWS // _SC_BR


def _tc_body(*refs):
    p_refs = refs[:_W]
    t_refs = refs[_W:2 * _W]
    out_ref = refs[2 * _W]
    err_acc, cnt_acc = refs[2 * _W + 1], refs[2 * _W + 2]
    i = pl.program_id(0)

    @pl.when(i == 0)
    def _init():
        err_acc[...] = jnp.zeros_like(err_acc)
        cnt_acc[...] = jnp.zeros_like(cnt_acc)

    e = jnp.zeros((1, _C), jnp.float32)
    c = jnp.zeros((1, _C), jnp.float32)
    for p_ref, t_ref in zip(p_refs, t_refs):
        p = p_ref[...]
        t = t_ref[...]
        valid = t > 0.0
        err = jnp.where(valid, jnp.abs(p - t), 0.0)
        cnt = jnp.where(valid, 1.0, 0.0)
        e += jnp.sum(err, axis=0, keepdims=True)
        c += jnp.sum(cnt, axis=0, keepdims=True)
    err_acc[...] += e
    cnt_acc[...] += c

    @pl.when(i == pl.num_programs(0) - 1)
    def _fini():
        out_ref[0, 0] = jnp.sum(err_acc[...])
        out_ref[0, 1] = jnp.sum(cnt_acc[...])


def _tc_partial(p, t):
    steps = _R_TC // _W // _BLK
    specs = [
        pl.BlockSpec((_BLK, _C), lambda i, w=w: (i + w * steps, 0))
        for w in range(_W)
    ]
    return pl.pallas_call(
        _tc_body,
        grid=(steps,),
        in_specs=specs + specs,
        out_specs=pl.BlockSpec(memory_space=pltpu.SMEM),
        out_shape=jax.ShapeDtypeStruct((1, 2), jnp.float32),
        scratch_shapes=[
            pltpu.VMEM((1, _C), jnp.float32),
            pltpu.VMEM((1, _C), jnp.float32),
        ],
    )(*([p] * _W + [t] * _W))


def _sc_kernel_body(p_hbm, t_hbm, out_hbm,
                    p0, t0, p1, t1, e_acc, c_acc, s_p0, s_t0, s_p1, s_t1):
    wid = lax.axis_index("s") * 2 + lax.axis_index("c")
    base = _R_TC + wid * _SC_ROWS

    e_acc[...] = jnp.zeros((_LANES,), jnp.float32)
    c_acc[...] = jnp.zeros((_LANES,), jnp.float32)

    bufs = ((p0, t0, s_p0, s_t0), (p1, t1, s_p1, s_t1))

    def start(b):
        pb, tb, sp, st = bufs[b % 2]
        rs = base + b * _SC_BR
        hp = pltpu.async_copy(p_hbm.at[pl.ds(rs, _SC_BR)], pb, sp)
        ht = pltpu.async_copy(t_hbm.at[pl.ds(rs, _SC_BR)], tb, st)
        return hp, ht

    def compute(b):
        pb, tb, _, _ = bufs[b % 2]

        @pl.loop(0, _SC_BR)
        def _(r):
            for c in range(0, _C, _LANES):
                sl = pl.ds(c, _LANES)
                p = pb[r, sl]
                t = tb[r, sl]
                valid = t > 0.0
                err = jnp.where(valid, jnp.abs(p - t), 0.0)
                cnt = jnp.where(valid, 1.0, 0.0)
                plsc.addupdate(e_acc.at[:], err)
                plsc.addupdate(c_acc.at[:], cnt)

    handles = start(0)
    for b in range(_SC_NB):
        nxt = start(b + 1) if b + 1 < _SC_NB else None
        handles[0].wait()
        handles[1].wait()
        compute(b)
        handles = nxt

    pltpu.sync_copy(e_acc, out_hbm.at[0, wid])
    pltpu.sync_copy(c_acc, out_hbm.at[1, wid])


def _sc_partial(p, t):
    mesh = plsc.VectorSubcoreMesh(core_axis_name="c", subcore_axis_name="s")
    k = pl.kernel(
        _sc_kernel_body,
        out_type=jax.ShapeDtypeStruct((2, _NW, _LANES), jnp.float32),
        mesh=mesh,
        compiler_params=pltpu.CompilerParams(use_tc_tiling_on_sc=True),
        scratch_types=[
            pltpu.VMEM((_SC_BR, _C), jnp.float32),
            pltpu.VMEM((_SC_BR, _C), jnp.float32),
            pltpu.VMEM((_SC_BR, _C), jnp.float32),
            pltpu.VMEM((_SC_BR, _C), jnp.float32),
            pltpu.VMEM((_LANES,), jnp.float32),
            pltpu.VMEM((_LANES,), jnp.float32),
            pltpu.SemaphoreType.DMA,
            pltpu.SemaphoreType.DMA,
            pltpu.SemaphoreType.DMA,
            pltpu.SemaphoreType.DMA,
        ],
    )
    return k(p, t)


def kernel(pred, target):
    p = pred.reshape(_R, _C)
    t = target.reshape(_R, _C)
    tc = _tc_partial(p, t)
    sc = _sc_partial(p, t)
    s = tc[0, 0] + jnp.sum(sc[0])
    n = tc[0, 1] + jnp.sum(sc[1])
    mae = s / jnp.maximum(n, 1.0)
    return jnp.where(n < 10.0, jnp.float32(-1.0), mae)


# speedup vs baseline: 2.0945x; 1.1161x over previous
import jax
import jax.numpy as jnp
from jax import lax
from jax.experimental import pallas as pl
from jax.experimental.pallas import tpu as pltpu
from jax.experimental.pallas import tpu_sc as plsc

_R = 8192
_C = 512

_R_SC = 2048
_R_TC = _R - _R_SC

_W = 4
_BLK = 256

_NW = 32
_LANES = 16
_SC_ROWS = _R_SC // _NW
_SC_BR = 16
_SC_NB = _SC_ROWS // _SC_BR


def _tc_body(*refs):
    p_refs = refs[:_W]
    t_refs = refs[_W:2 * _W]
    out_ref = refs[2 * _W]
    err_acc, cnt_acc = refs[2 * _W + 1], refs[2 * _W + 2]
    i = pl.program_id(0)

    @pl.when(i == 0)
    def _init():
        err_acc[...] = jnp.zeros_like(err_acc)
        cnt_acc[...] = jnp.zeros_like(cnt_acc)

    e = jnp.zeros((1, _C), jnp.float32)
    c = jnp.zeros((1, _C), jnp.float32)
    for p_ref, t_ref in zip(p_refs, t_refs):
        p = p_ref[...]
        t = t_ref[...]
        valid = t > 0.0
        err = jnp.where(valid, jnp.abs(p - t), 0.0)
        cnt = jnp.where(valid, 1.0, 0.0)
        e += jnp.sum(err, axis=0, keepdims=True)
        c += jnp.sum(cnt, axis=0, keepdims=True)
    err_acc[...] += e
    cnt_acc[...] += c

    @pl.when(i == pl.num_programs(0) - 1)
    def _fini():
        out_ref[0, 0] = jnp.sum(err_acc[...])
        out_ref[0, 1] = jnp.sum(cnt_acc[...])


def _tc_partial(p, t):
    steps = _R_TC // _W // _BLK
    specs = [
        pl.BlockSpec((_BLK, _C), lambda i, w=w: (i + w * steps, 0))
        for w in range(_W)
    ]
    return pl.pallas_call(
        _tc_body,
        grid=(steps,),
        in_specs=specs + specs,
        out_specs=pl.BlockSpec(memory_space=pltpu.SMEM),
        out_shape=jax.ShapeDtypeStruct((1, 2), jnp.float32),
        scratch_shapes=[
            pltpu.VMEM((1, _C), jnp.float32),
            pltpu.VMEM((1, _C), jnp.float32),
        ],
    )(*([p] * _W + [t] * _W))


def _sc_kernel_body(p_hbm, t_hbm, out_hbm,
                    p0, t0, p1, t1, e_acc, c_acc, s_p0, s_t0, s_p1, s_t1):
    wid = lax.axis_index("s") * 2 + lax.axis_index("c")
    base = _R_TC + wid * _SC_ROWS

    bufs = ((p0, t0, s_p0, s_t0), (p1, t1, s_p1, s_t1))
    _NA = 4

    def start(b):
        pb, tb, sp, st = bufs[b % 2]
        rs = base + b * _SC_BR
        hp = pltpu.async_copy(p_hbm.at[pl.ds(rs, _SC_BR)], pb, sp)
        ht = pltpu.async_copy(t_hbm.at[pl.ds(rs, _SC_BR)], tb, st)
        return hp, ht

    def compute(b, accs):
        pb, tb, _, _ = bufs[b % 2]

        def row_body(r, carry):
            es = list(carry[:_NA])
            cs = list(carry[_NA:])
            for ci, c in enumerate(range(0, _C, _LANES)):
                sl = pl.ds(c, _LANES)
                p = pb[r, sl]
                t = tb[r, sl]
                valid = t > 0.0
                j = ci % _NA
                es[j] = es[j] + jnp.where(valid, jnp.abs(p - t), 0.0)
                cs[j] = cs[j] + jnp.where(valid, 1.0, 0.0)
            return tuple(es) + tuple(cs)

        return lax.fori_loop(0, _SC_BR, row_body, accs)

    zero = jnp.zeros((_LANES,), jnp.float32)
    accs = (zero,) * 8
    handles = start(0)
    for b in range(_SC_NB):
        nxt = start(b + 1) if b + 1 < _SC_NB else None
        handles[0].wait()
        handles[1].wait()
        accs = compute(b, accs)
        handles = nxt

    e_acc[...] = accs[0] + accs[1] + accs[2] + accs[3]
    c_acc[...] = accs[4] + accs[5] + accs[6] + accs[7]
    pltpu.sync_copy(e_acc, out_hbm.at[0, wid])
    pltpu.sync_copy(c_acc, out_hbm.at[1, wid])


def _sc_partial(p, t):
    mesh = plsc.VectorSubcoreMesh(core_axis_name="c", subcore_axis_name="s")
    k = pl.kernel(
        _sc_kernel_body,
        out_type=jax.ShapeDtypeStruct((2, _NW, _LANES), jnp.float32),
        mesh=mesh,
        compiler_params=pltpu.CompilerParams(use_tc_tiling_on_sc=True),
        scratch_types=[
            pltpu.VMEM((_SC_BR, _C), jnp.float32),
            pltpu.VMEM((_SC_BR, _C), jnp.float32),
            pltpu.VMEM((_SC_BR, _C), jnp.float32),
            pltpu.VMEM((_SC_BR, _C), jnp.float32),
            pltpu.VMEM((_LANES,), jnp.float32),
            pltpu.VMEM((_LANES,), jnp.float32),
            pltpu.SemaphoreType.DMA,
            pltpu.SemaphoreType.DMA,
            pltpu.SemaphoreType.DMA,
            pltpu.SemaphoreType.DMA,
        ],
    )
    return k(p, t)


def kernel(pred, target):
    p = pred.reshape(_R, _C)
    t = target.reshape(_R, _C)
    tc = _tc_partial(p, t)
    sc = _sc_partial(p, t)
    s = tc[0, 0] + jnp.sum(sc[0])
    n = tc[0, 1] + jnp.sum(sc[1])
    mae = s / jnp.maximum(n, 1.0)
    return jnp.where(n < 10.0, jnp.float32(-1.0), mae)


# speedup vs baseline: 2.3118x; 1.1037x over previous
import jax
import jax.numpy as jnp
from jax import lax
from jax.experimental import pallas as pl
from jax.experimental.pallas import tpu as pltpu
from jax.experimental.pallas import tpu_sc as plsc

_R = 8192
_C = 512

_R_SC = 512
_R_TC = _R - _R_SC

_W = 4
_BLK = 240

_NW = 32
_LANES = 16
_SC_ROWS = _R_SC // _NW
_SC_BR = 8
_SC_NB = _SC_ROWS // _SC_BR


def _tc_body(*refs):
    p_refs = refs[:_W]
    t_refs = refs[_W:2 * _W]
    out_ref = refs[2 * _W]
    err_acc, cnt_acc = refs[2 * _W + 1], refs[2 * _W + 2]
    i = pl.program_id(0)

    @pl.when(i == 0)
    def _init():
        err_acc[...] = jnp.zeros_like(err_acc)
        cnt_acc[...] = jnp.zeros_like(cnt_acc)

    e = jnp.zeros((1, _C), jnp.float32)
    c = jnp.zeros((1, _C), jnp.float32)
    for p_ref, t_ref in zip(p_refs, t_refs):
        p = p_ref[...]
        t = t_ref[...]
        valid = t > 0.0
        err = jnp.where(valid, jnp.abs(p - t), 0.0)
        cnt = jnp.where(valid, 1.0, 0.0)
        e += jnp.sum(err, axis=0, keepdims=True)
        c += jnp.sum(cnt, axis=0, keepdims=True)
    err_acc[...] += e
    cnt_acc[...] += c

    @pl.when(i == pl.num_programs(0) - 1)
    def _fini():
        out_ref[0, 0] = jnp.sum(err_acc[...])
        out_ref[0, 1] = jnp.sum(cnt_acc[...])


def _tc_partial(p, t):
    steps = _R_TC // _W // _BLK
    specs = [
        pl.BlockSpec((_BLK, _C), lambda i, w=w: (i + w * steps, 0))
        for w in range(_W)
    ]
    return pl.pallas_call(
        _tc_body,
        grid=(steps,),
        in_specs=specs + specs,
        out_specs=pl.BlockSpec(memory_space=pltpu.SMEM),
        out_shape=jax.ShapeDtypeStruct((1, 2), jnp.float32),
        scratch_shapes=[
            pltpu.VMEM((1, _C), jnp.float32),
            pltpu.VMEM((1, _C), jnp.float32),
        ],
    )(*([p] * _W + [t] * _W))


def _sc_kernel_body(p_hbm, t_hbm, out_hbm,
                    p0, t0, p1, t1, e_acc, c_acc, s_p0, s_t0, s_p1, s_t1):
    wid = lax.axis_index("s") * 2 + lax.axis_index("c")
    base = _R_TC + wid * _SC_ROWS

    bufs = ((p0, t0, s_p0, s_t0), (p1, t1, s_p1, s_t1))
    _NA = 4

    def start(b):
        pb, tb, sp, st = bufs[b % 2]
        rs = base + b * _SC_BR
        hp = pltpu.async_copy(p_hbm.at[pl.ds(rs, _SC_BR)], pb, sp)
        ht = pltpu.async_copy(t_hbm.at[pl.ds(rs, _SC_BR)], tb, st)
        return hp, ht

    def compute(b, accs):
        pb, tb, _, _ = bufs[b % 2]
        cb_per_row = _C // (_NA * _LANES)

        def body(i, carry):
            r = i // cb_per_row
            cb = (i - r * cb_per_row) * (_NA * _LANES)
            es = list(carry[:_NA])
            cs = list(carry[_NA:])
            for j in range(_NA):
                sl = pl.ds(cb + j * _LANES, _LANES)
                p = pb[r, sl]
                t = tb[r, sl]
                valid = t > 0.0
                es[j] = es[j] + jnp.where(valid, jnp.abs(p - t), 0.0)
                cs[j] = cs[j] + jnp.where(valid, 1.0, 0.0)
            return tuple(es) + tuple(cs)

        return lax.fori_loop(0, _SC_BR * cb_per_row, body, accs)

    zero = jnp.zeros((_LANES,), jnp.float32)
    accs = (zero,) * 8
    handles = start(0)
    for b in range(_SC_NB):
        nxt = start(b + 1) if b + 1 < _SC_NB else None
        handles[0].wait()
        handles[1].wait()
        accs = compute(b, accs)
        handles = nxt

    e_acc[...] = accs[0] + accs[1] + accs[2] + accs[3]
    c_acc[...] = accs[4] + accs[5] + accs[6] + accs[7]
    pltpu.sync_copy(e_acc, out_hbm.at[0, wid])
    pltpu.sync_copy(c_acc, out_hbm.at[1, wid])


def _sc_partial(p, t):
    mesh = plsc.VectorSubcoreMesh(core_axis_name="c", subcore_axis_name="s")
    k = pl.kernel(
        _sc_kernel_body,
        out_type=jax.ShapeDtypeStruct((2, _NW, _LANES), jnp.float32),
        mesh=mesh,
        compiler_params=pltpu.CompilerParams(use_tc_tiling_on_sc=True),
        scratch_types=[
            pltpu.VMEM((_SC_BR, _C), jnp.float32),
            pltpu.VMEM((_SC_BR, _C), jnp.float32),
            pltpu.VMEM((_SC_BR, _C), jnp.float32),
            pltpu.VMEM((_SC_BR, _C), jnp.float32),
            pltpu.VMEM((_LANES,), jnp.float32),
            pltpu.VMEM((_LANES,), jnp.float32),
            pltpu.SemaphoreType.DMA,
            pltpu.SemaphoreType.DMA,
            pltpu.SemaphoreType.DMA,
            pltpu.SemaphoreType.DMA,
        ],
    )
    return k(p, t)


def kernel(pred, target):
    p = pred.reshape(_R, _C)
    t = target.reshape(_R, _C)
    tc = _tc_partial(p, t)
    sc = _sc_partial(p, t)
    s = tc[0, 0] + jnp.sum(sc[0])
    n = tc[0, 1] + jnp.sum(sc[1])
    mae = s / jnp.maximum(n, 1.0)
    return jnp.where(n < 10.0, jnp.float32(-1.0), mae)


# speedup vs baseline: 5.7935x; 2.5060x over previous
import jax
import jax.numpy as jnp
from jax.experimental import pallas as pl
from jax.experimental.pallas import tpu as pltpu

_R = 8192
_C = 512
_W = 8
_BLK = 128


def _mae_body(*refs):
    p_refs = refs[:_W]
    t_refs = refs[_W:2 * _W]
    out_ref = refs[2 * _W]
    err_acc, cnt_acc = refs[2 * _W + 1], refs[2 * _W + 2]
    i = pl.program_id(0)

    @pl.when(i == 0)
    def _init():
        err_acc[...] = jnp.zeros_like(err_acc)
        cnt_acc[...] = jnp.zeros_like(cnt_acc)

    e = jnp.zeros((1, _C), jnp.float32)
    c = jnp.zeros((1, _C), jnp.float32)
    for p_ref, t_ref in zip(p_refs, t_refs):
        p = p_ref[...]
        t = t_ref[...]
        valid = t > 0.0
        err = jnp.where(valid, jnp.abs(p - t), 0.0)
        cnt = jnp.where(valid, 1.0, 0.0)
        e += jnp.sum(err, axis=0, keepdims=True)
        c += jnp.sum(cnt, axis=0, keepdims=True)
    err_acc[...] += e
    cnt_acc[...] += c

    @pl.when(i == pl.num_programs(0) - 1)
    def _fini():
        s = jnp.sum(err_acc[...])
        n = jnp.sum(cnt_acc[...])
        mae = s / jnp.maximum(n, 1.0)
        out_ref[0, 0] = jnp.where(n < 10.0, jnp.float32(-1.0), mae)


def kernel(pred, target):
    p = pred.reshape(_R, _C)
    t = target.reshape(_R, _C)
    steps = _R // _W // _BLK
    specs = [
        pl.BlockSpec((_BLK, _C), lambda i, w=w: (i + w * steps, 0))
        for w in range(_W)
    ]
    out = pl.pallas_call(
        _mae_body,
        grid=(steps,),
        in_specs=specs + specs,
        out_specs=pl.BlockSpec(memory_space=pltpu.SMEM),
        out_shape=jax.ShapeDtypeStruct((1, 1), jnp.float32),
        scratch_shapes=[
            pltpu.VMEM((1, _C), jnp.float32),
            pltpu.VMEM((1, _C), jnp.float32),
        ],
    )(*([p] * _W + [t] * _W))
    return out[0, 0]


# speedup vs baseline: 6.3565x; 1.0972x over previous
import jax
import jax.numpy as jnp
from jax.experimental import pallas as pl
from jax.experimental.pallas import tpu as pltpu

_R = 8192
_C = 512
_W = 4
_BLK = 512


def _mae_body(*refs):
    p_refs = refs[:_W]
    t_refs = refs[_W:2 * _W]
    out_ref = refs[2 * _W]
    err_acc, cnt_acc = refs[2 * _W + 1], refs[2 * _W + 2]
    i = pl.program_id(0)

    @pl.when(i == 0)
    def _init():
        err_acc[...] = jnp.zeros_like(err_acc)
        cnt_acc[...] = jnp.zeros_like(cnt_acc)

    e = jnp.zeros((1, _C), jnp.float32)
    c = jnp.zeros((1, _C), jnp.float32)
    for p_ref, t_ref in zip(p_refs, t_refs):
        p = p_ref[...]
        t = t_ref[...]
        valid = t > 0.0
        err = jnp.where(valid, jnp.abs(p - t), 0.0)
        cnt = jnp.where(valid, 1.0, 0.0)
        e += jnp.sum(err, axis=0, keepdims=True)
        c += jnp.sum(cnt, axis=0, keepdims=True)
    err_acc[...] += e
    cnt_acc[...] += c

    @pl.when(i == pl.num_programs(0) - 1)
    def _fini():
        s = jnp.sum(err_acc[...])
        n = jnp.sum(cnt_acc[...])
        mae = s / jnp.maximum(n, 1.0)
        out_ref[0, 0] = jnp.where(n < 10.0, jnp.float32(-1.0), mae)


def kernel(pred, target):
    p = pred.reshape(_R, _C)
    t = target.reshape(_R, _C)
    steps = _R // _W // _BLK
    specs = [
        pl.BlockSpec((_BLK, _C), lambda i, w=w: (i + w * steps, 0))
        for w in range(_W)
    ]
    out = pl.pallas_call(
        _mae_body,
        grid=(steps,),
        in_specs=specs + specs,
        out_specs=pl.BlockSpec(memory_space=pltpu.SMEM),
        out_shape=jax.ShapeDtypeStruct((1, 1), jnp.float32),
        scratch_shapes=[
            pltpu.VMEM((1, _C), jnp.float32),
            pltpu.VMEM((1, _C), jnp.float32),
        ],
    )(*([p] * _W + [t] * _W))
    return out[0, 0]
